# baseline (speedup 1.0000x reference)
import dataclasses
import functools

import jax
import jax.numpy as jnp
from jax import lax
from jax.experimental import pallas as pl
from jax.experimental.pallas import tpu as pltpu
from jax.experimental.pallas import tpu_sc as plsc

N = 10000
NP = 10240
E = 320000
EE = E + N
NC, NS = 2, 16
NW = NC * NS
CH = 81
EW = CH * 128
EEP = EW * NW
NG = 64
F_IN = 128
H1, P1 = 64, 80
H2, P2 = 128, 144
NBLK = 2048
GRID_N = NP // NBLK
STRIPE = NP // NS

_mesh = plsc.VectorSubcoreMesh(
    core_axis_name="c", subcore_axis_name="s", num_cores=NC, num_subcores=NS)

_sc_params = pltpu.CompilerParams()
if "needs_layout_passes" in pltpu.CompilerParams.__dataclass_fields__:
    _sc_params = dataclasses.replace(_sc_params, needs_layout_passes=False)
if "use_tc_tiling_on_sc" in pltpu.CompilerParams.__dataclass_fields__:
    _sc_params = dataclasses.replace(_sc_params, use_tc_tiling_on_sc=False)


def _bf16_bits(x):
    b = jax.lax.bitcast_convert_type(x, jnp.int32)
    return (b + jnp.int32(0x7FFF) + ((b >> 16) & jnp.int32(1))) >> 16


def _interleave_bf16(h64):
    parts = []
    for g in range(2):
        blk = h64[:, 32 * g:32 * g + 32]
        lo = _bf16_bits(blk[:, :16]) & jnp.int32(0xFFFF)
        hi = _bf16_bits(blk[:, 16:])
        parts.append(lo | (hi << 16))
    return jnp.concatenate(parts, axis=1)


def _dense_body(x_ref, w_ref, asrc_ref, adst_ref, *out_refs):
    h = jnp.dot(x_ref[...], w_ref[...], preferred_element_type=jnp.float32)
    nh = len(out_refs) - 2
    for k in range(nh):
        out_refs[k][...] = _interleave_bf16(h[:, k * 64:(k + 1) * 64])
    out_refs[nh][...] = jnp.sum(h * asrc_ref[...], axis=1)
    out_refs[nh + 1][...] = jnp.sum(h * adst_ref[...], axis=1)


def _dense(x, W, a_src, a_dst, H):
    F = x.shape[1]
    nh = H // 64
    return pl.pallas_call(
        _dense_body,
        grid=(GRID_N,),
        in_specs=[
            pl.BlockSpec((NBLK, F), lambda i: (i, 0)),
            pl.BlockSpec((F, H), lambda i: (0, 0)),
            pl.BlockSpec((1, H), lambda i: (0, 0)),
            pl.BlockSpec((1, H), lambda i: (0, 0)),
        ],
        out_specs=[pl.BlockSpec((NBLK, 32), lambda i: (i, 0))] * nh
        + [pl.BlockSpec((NBLK,), lambda i: (i,))] * 2,
        out_shape=[jax.ShapeDtypeStruct((NP, 32), jnp.int32)] * nh
        + [jax.ShapeDtypeStruct((NP,), jnp.float32)] * 2,
    )(x, W, a_src.reshape(1, H), a_dst.reshape(1, H))


@functools.partial(
    pl.kernel,
    out_type=(jax.ShapeDtypeStruct((EEP,), jnp.float32),
              jax.ShapeDtypeStruct((NW, NP), jnp.float32)),
    mesh=_mesh,
    scratch_types=[
        pltpu.VMEM((NP,), jnp.float32),
        pltpu.VMEM((NP,), jnp.float32),
        pltpu.VMEM((EW,), jnp.int32),
        pltpu.VMEM((EW,), jnp.int32),
        pltpu.VMEM((EW,), jnp.float32),
        pltpu.VMEM((NP,), jnp.float32),
    ],
    compiler_params=_sc_params,
)
def _edge_w(as_hbm, ad_hbm, src_hbm, dst_hbm, w_hbm, den_hbm, as_v, ad_v,
            src_v, dst_v, w_v, den_v):
    wid = lax.axis_index("s") * NC + lax.axis_index("c")
    base = wid * EW
    pltpu.sync_copy(as_hbm, as_v)
    pltpu.sync_copy(ad_hbm, ad_v)
    pltpu.sync_copy(src_hbm.at[pl.ds(base, EW)], src_v)
    pltpu.sync_copy(dst_hbm.at[pl.ds(base, EW)], dst_v)

    @pl.loop(0, NP, step=16)
    def _(k):
        den_v[pl.ds(k, 16)] = jnp.zeros((16,), jnp.float32)

    @pl.loop(0, EW, step=16)
    def _(k):
        s16 = src_v[pl.ds(k, 16)]
        d16 = dst_v[pl.ds(k, 16)]
        gs = plsc.load_gather(as_v, [s16])
        gd = plsc.load_gather(ad_v, [d16])
        e = gs + gd
        e = jnp.where(e >= 0.0, e, e * jnp.float32(0.2))
        w = jnp.exp(e)
        gi = base + k + lax.iota(jnp.int32, 16)
        w = jnp.where(gi < EE, w, jnp.float32(0.0))
        w_v[pl.ds(k, 16)] = w
        plsc.addupdate_scatter(den_v, [d16], w)

    pltpu.sync_copy(w_v, w_hbm.at[pl.ds(base, EW)])
    pltpu.sync_copy(den_v, den_hbm.at[wid])


def _edge_pass(h64, w, src3, dst3, zeros, P=64):
    @functools.partial(
        pl.kernel,
        out_type=jax.ShapeDtypeStruct((NC, NP, P), jnp.float32),
        mesh=_mesh,
        scratch_types=[
            pltpu.VMEM((CH, 128), jnp.int32),
            pltpu.VMEM((CH, 128), jnp.int32),
            pltpu.VMEM((EW,), jnp.float32),
            pltpu.VMEM((128, 32), jnp.int32),
            pltpu.VMEM((128, 32), jnp.int32),
            pltpu.VMEM((128, P), jnp.float32),
            pltpu.VMEM((128, P), jnp.float32),
            pltpu.VMEM_SHARED((NP, P), jnp.float32),
            pltpu.SemaphoreType.DMA,
            pltpu.SemaphoreType.DMA,
            pltpu.SemaphoreType.DMA,
            pltpu.SemaphoreType.DMA,
        ],
        compiler_params=_sc_params,
    )
    def kern(h_hbm, w_hbm, src3_hbm, dst3_hbm, z_hbm, out_hbm,
             src_v, dst_v, w_v, gb0, gb1, sb0, sb1, acc,
             gsem0, gsem1, ssem0, ssem1):
        cid = lax.axis_index("c")
        sid = lax.axis_index("s")
        wid = sid * NC + cid
        base = wid * EW
        gbufs = (gb0, gb1)
        sbufs = (sb0, sb1)
        gsems = (gsem0, gsem1)
        ssems = (ssem0, ssem1)

        @pl.loop(0, STRIPE // 128)
        def _(j):
            pltpu.sync_copy(z_hbm, acc.at[pl.ds(sid * STRIPE + j * 128, 128)])

        pltpu.sync_copy(src3_hbm.at[wid], src_v)
        pltpu.sync_copy(dst3_hbm.at[wid], dst_v)
        pltpu.sync_copy(w_hbm.at[pl.ds(base, EW)], w_v)
        plsc.subcore_barrier()

        pltpu.async_copy(h_hbm.at[src_v.at[0]], gb0, gsem0)

        @pl.loop(0, CH, step=2)
        def _(c):
            for par in (0, 1):
                cc = c + par

                @pl.when(cc < CH)
                def _():
                    gb, sb = gbufs[par], sbufs[par]
                    pltpu.make_async_copy(
                        h_hbm.at[src_v.at[cc]], gb, gsems[par]).wait()

                    @pl.when(cc + 1 < CH)
                    def _():
                        pltpu.async_copy(h_hbm.at[src_v.at[cc + 1]],
                                         gbufs[1 - par], gsems[1 - par])

                    @pl.when(cc >= 2)
                    def _():
                        pltpu.make_async_copy(
                            sb, acc.at[dst_v.at[cc]], ssems[par]).wait()

                    @pl.loop(0, 128)
                    def _(j):
                        idx16 = jnp.full((16,), cc * 128 + j, jnp.int32)
                        wv = plsc.load_gather(w_v, [idx16])
                        for t in range(2):
                            bits = gb[j, pl.ds(t * 16, 16)]
                            lo = plsc.bitcast(bits << 16, jnp.float32)
                            hi = plsc.bitcast(bits & jnp.int32(-65536),
                                              jnp.float32)
                            sb[j, pl.ds(t * 32, 16)] = lo * wv
                            sb[j, pl.ds(t * 32 + 16, 16)] = hi * wv

                    pltpu.async_copy(sb, acc.at[dst_v.at[cc]], ssems[par],
                                     add=True)

        pltpu.make_async_copy(sb0, acc.at[dst_v.at[0]], ssem0).wait()
        pltpu.make_async_copy(sb1, acc.at[dst_v.at[1]], ssem1).wait()
        plsc.subcore_barrier()
        pltpu.sync_copy(acc.at[pl.ds(sid * STRIPE, STRIPE)],
                        out_hbm.at[cid].at[pl.ds(sid * STRIPE, STRIPE)])

    return kern(h64, w, src3, dst3, zeros)


def _layer_out(p_refs, den_ref, b_ref):
    den = jnp.sum(den_ref[...], axis=0)[:, None] + jnp.float32(1e-16)
    parts = [pr[0] + pr[1] for pr in p_refs]
    return jnp.maximum(jnp.concatenate(parts, axis=1) / den + b_ref[...], 0.0)


def _comb_dense_body(p_ref, den_ref, b_ref, w_ref, asrc_ref, adst_ref,
                     hA_ref, hB_ref, as_ref, ad_ref):
    x2 = _layer_out([p_ref], den_ref, b_ref)
    h = jnp.dot(x2, w_ref[...], preferred_element_type=jnp.float32)
    hA_ref[...] = _interleave_bf16(h[:, :64])
    hB_ref[...] = _interleave_bf16(h[:, 64:])
    as_ref[...] = jnp.sum(h * asrc_ref[...], axis=1)
    ad_ref[...] = jnp.sum(h * adst_ref[...], axis=1)


def _comb_dense(partials, den, b, W, a_src, a_dst):
    return pl.pallas_call(
        _comb_dense_body,
        grid=(GRID_N,),
        in_specs=[
            pl.BlockSpec((NC, NBLK, 64), lambda i: (0, i, 0)),
            pl.BlockSpec((NW, NBLK), lambda i: (0, i)),
            pl.BlockSpec((1, H1), lambda i: (0, 0)),
            pl.BlockSpec((H1, H2), lambda i: (0, 0)),
            pl.BlockSpec((1, H2), lambda i: (0, 0)),
            pl.BlockSpec((1, H2), lambda i: (0, 0)),
        ],
        out_specs=[pl.BlockSpec((NBLK, 32), lambda i: (i, 0))] * 2
        + [pl.BlockSpec((NBLK,), lambda i: (i,))] * 2,
        out_shape=[jax.ShapeDtypeStruct((NP, 32), jnp.int32)] * 2
        + [jax.ShapeDtypeStruct((NP,), jnp.float32)] * 2,
    )(partials, den, b.reshape(1, H1), W, a_src.reshape(1, H2),
      a_dst.reshape(1, H2))


def _comb_pool_body(pA_ref, pB_ref, den_ref, b_ref, batch_ref, wfc_ref,
                    bfc_ref, o_ref, sums, cnt):
    i = pl.program_id(0)

    @pl.when(i == 0)
    def _():
        sums[...] = jnp.zeros_like(sums)
        cnt[...] = jnp.zeros_like(cnt)

    x3 = _layer_out([pA_ref, pB_ref], den_ref, b_ref)
    bb = batch_ref[...]
    M = (lax.broadcasted_iota(jnp.int32, (NG, NBLK), 0)
         == bb[None, :]).astype(jnp.float32)
    sums[...] += jnp.dot(M, x3, preferred_element_type=jnp.float32)
    cnt[...] += jnp.sum(M, axis=1, keepdims=True)

    @pl.when(i == pl.num_programs(0) - 1)
    def _():
        pooled = sums[...] / jnp.maximum(cnt[...], 1.0)
        o_ref[...] = jnp.dot(pooled, wfc_ref[...],
                             preferred_element_type=jnp.float32) + bfc_ref[...]


def _comb_pool(pA, pB, den, b, batch_p, W_fc, b_fc):
    return pl.pallas_call(
        _comb_pool_body,
        grid=(GRID_N,),
        in_specs=[
            pl.BlockSpec((NC, NBLK, 64), lambda i: (0, i, 0)),
            pl.BlockSpec((NC, NBLK, 64), lambda i: (0, i, 0)),
            pl.BlockSpec((NW, NBLK), lambda i: (0, i)),
            pl.BlockSpec((1, H2), lambda i: (0, 0)),
            pl.BlockSpec((NBLK,), lambda i: (i,)),
            pl.BlockSpec((H2, NG), lambda i: (0, 0)),
            pl.BlockSpec((1, NG), lambda i: (0, 0)),
        ],
        out_specs=pl.BlockSpec((NG, NG), lambda i: (0, 0)),
        out_shape=jax.ShapeDtypeStruct((NG, NG), jnp.float32),
        scratch_shapes=[
            pltpu.VMEM((NG, H2), jnp.float32),
            pltpu.VMEM((NG, 1), jnp.float32),
        ],
    )(pA, pB, den, b.reshape(1, H2), batch_p, W_fc, b_fc.reshape(1, NG))


def kernel(x, edge_index, batch, W1, a1_src, a1_dst, b1, W2, a2_src, a2_dst,
           b2, W_fc, b_fc):
    loop = jnp.arange(N, dtype=edge_index.dtype)
    src = jnp.concatenate([edge_index[0], loop])
    dst = jnp.concatenate([edge_index[1], loop])
    pad = jnp.zeros((EEP - EE,), jnp.int32)
    srcp = jnp.concatenate([src, pad])
    dstp = jnp.concatenate([dst, pad])
    src3 = srcp.reshape(NW, CH, 128)
    dst3 = dstp.reshape(NW, CH, 128)
    xp = jnp.pad(x, ((0, NP - N), (0, 0)))
    batch_p = jnp.pad(batch, (0, NP - N), constant_values=NG)
    zeros = jnp.zeros((128, 64), jnp.float32)

    h1, as1, ad1 = _dense(xp, W1, a1_src, a1_dst, H1)
    w1, den1 = _edge_w(as1, ad1, srcp, dstp)
    p1 = _edge_pass(h1, w1, src3, dst3, zeros)
    hA, hB, as2, ad2 = _comb_dense(p1, den1, b1, W2, a2_src, a2_dst)
    w2, den2 = _edge_w(as2, ad2, srcp, dstp)
    pA = _edge_pass(hA, w2, src3, dst3, zeros)
    pB = _edge_pass(hB, w2, src3, dst3, zeros)
    return _comb_pool(pA, pB, den2, b2, batch_p, W_fc, b_fc)


# speedup vs baseline: 1.1525x; 1.1525x over previous
import dataclasses
import functools

import jax
import jax.numpy as jnp
from jax import lax
from jax.experimental import pallas as pl
from jax.experimental.pallas import tpu as pltpu
from jax.experimental.pallas import tpu_sc as plsc

N = 10000
NP = 10240
E = 320000
EE = E + N
NC, NS = 2, 16
NW = NC * NS
CH = 81
EW = CH * 128
EEP = EW * NW
NG = 64
F_IN = 128
H1, P1 = 64, 80
H2, P2 = 128, 144
NBLK = 2048
GRID_N = NP // NBLK
STRIPE = NP // NS

_mesh = plsc.VectorSubcoreMesh(
    core_axis_name="c", subcore_axis_name="s", num_cores=NC, num_subcores=NS)

_sc_params = pltpu.CompilerParams()
if "needs_layout_passes" in pltpu.CompilerParams.__dataclass_fields__:
    _sc_params = dataclasses.replace(_sc_params, needs_layout_passes=False)
if "use_tc_tiling_on_sc" in pltpu.CompilerParams.__dataclass_fields__:
    _sc_params = dataclasses.replace(_sc_params, use_tc_tiling_on_sc=False)


def _bf16_bits(x):
    b = jax.lax.bitcast_convert_type(x, jnp.int32)
    return (b + jnp.int32(0x7FFF) + ((b >> 16) & jnp.int32(1))) >> 16


def _interleave_bf16(h64):
    parts = []
    for g in range(2):
        blk = h64[:, 32 * g:32 * g + 32]
        lo = _bf16_bits(blk[:, :16]) & jnp.int32(0xFFFF)
        hi = _bf16_bits(blk[:, 16:])
        parts.append(lo | (hi << 16))
    return jnp.concatenate(parts, axis=1)


def _dense_body(x_ref, w_ref, asrc_ref, adst_ref, *out_refs):
    h = jnp.dot(x_ref[...], w_ref[...], preferred_element_type=jnp.float32)
    nh = len(out_refs) - 2
    for k in range(nh):
        out_refs[k][...] = h[:, k * 64:(k + 1) * 64]
    out_refs[nh][...] = jnp.sum(h * asrc_ref[...], axis=1)
    out_refs[nh + 1][...] = jnp.sum(h * adst_ref[...], axis=1)


def _dense(x, W, a_src, a_dst, H):
    F = x.shape[1]
    nh = H // 64
    return pl.pallas_call(
        _dense_body,
        grid=(GRID_N,),
        in_specs=[
            pl.BlockSpec((NBLK, F), lambda i: (i, 0)),
            pl.BlockSpec((F, H), lambda i: (0, 0)),
            pl.BlockSpec((1, H), lambda i: (0, 0)),
            pl.BlockSpec((1, H), lambda i: (0, 0)),
        ],
        out_specs=[pl.BlockSpec((NBLK, 64), lambda i: (i, 0))] * nh
        + [pl.BlockSpec((NBLK,), lambda i: (i,))] * 2,
        out_shape=[jax.ShapeDtypeStruct((NP, 64), jnp.float32)] * nh
        + [jax.ShapeDtypeStruct((NP,), jnp.float32)] * 2,
    )(x, W, a_src.reshape(1, H), a_dst.reshape(1, H))


@functools.partial(
    pl.kernel,
    out_type=(jax.ShapeDtypeStruct((EEP,), jnp.float32),
              jax.ShapeDtypeStruct((NW, NP), jnp.float32)),
    mesh=_mesh,
    scratch_types=[
        pltpu.VMEM((NP,), jnp.float32),
        pltpu.VMEM((NP,), jnp.float32),
        pltpu.VMEM((EW,), jnp.int32),
        pltpu.VMEM((EW,), jnp.int32),
        pltpu.VMEM((EW,), jnp.float32),
        pltpu.VMEM((NP,), jnp.float32),
    ],
    compiler_params=_sc_params,
)
def _edge_w(as_hbm, ad_hbm, src_hbm, dst_hbm, w_hbm, den_hbm, as_v, ad_v,
            src_v, dst_v, w_v, den_v):
    wid = lax.axis_index("s") * NC + lax.axis_index("c")
    base = wid * EW
    pltpu.sync_copy(as_hbm, as_v)
    pltpu.sync_copy(ad_hbm, ad_v)
    pltpu.sync_copy(src_hbm.at[pl.ds(base, EW)], src_v)
    pltpu.sync_copy(dst_hbm.at[pl.ds(base, EW)], dst_v)

    @pl.loop(0, NP, step=16)
    def _(k):
        den_v[pl.ds(k, 16)] = jnp.zeros((16,), jnp.float32)

    @pl.loop(0, EW, step=16)
    def _(k):
        s16 = src_v[pl.ds(k, 16)]
        d16 = dst_v[pl.ds(k, 16)]
        gs = plsc.load_gather(as_v, [s16])
        gd = plsc.load_gather(ad_v, [d16])
        e = gs + gd
        e = jnp.where(e >= 0.0, e, e * jnp.float32(0.2))
        w = jnp.exp(e)
        gi = base + k + lax.iota(jnp.int32, 16)
        w = jnp.where(gi < EE, w, jnp.float32(0.0))
        w_v[pl.ds(k, 16)] = w
        plsc.addupdate_scatter(den_v, [d16], w)

    pltpu.sync_copy(w_v, w_hbm.at[pl.ds(base, EW)])
    pltpu.sync_copy(den_v, den_hbm.at[wid])


def _edge_pass(h64, w, src3, dst3, zeros, P=64):
    @functools.partial(
        pl.kernel,
        out_type=jax.ShapeDtypeStruct((NC, NP, P), jnp.float32),
        mesh=_mesh,
        scratch_types=[
            pltpu.VMEM((CH, 128), jnp.int32),
            pltpu.VMEM((CH, 128), jnp.int32),
            pltpu.VMEM((EW,), jnp.float32),
            pltpu.VMEM((128, 64), jnp.float32),
            pltpu.VMEM((128, 64), jnp.float32),
            pltpu.VMEM((128, P), jnp.float32),
            pltpu.VMEM((128, P), jnp.float32),
            pltpu.VMEM_SHARED((NP, P), jnp.float32),
            pltpu.SemaphoreType.DMA,
            pltpu.SemaphoreType.DMA,
            pltpu.SemaphoreType.DMA,
            pltpu.SemaphoreType.DMA,
        ],
        compiler_params=_sc_params,
    )
    def kern(h_hbm, w_hbm, src3_hbm, dst3_hbm, z_hbm, out_hbm,
             src_v, dst_v, w_v, gb0, gb1, sb0, sb1, acc,
             gsem0, gsem1, ssem0, ssem1):
        cid = lax.axis_index("c")
        sid = lax.axis_index("s")
        wid = sid * NC + cid
        base = wid * EW
        gbufs = (gb0, gb1)
        sbufs = (sb0, sb1)
        gsems = (gsem0, gsem1)
        ssems = (ssem0, ssem1)

        @pl.loop(0, STRIPE // 128)
        def _(j):
            pltpu.sync_copy(z_hbm, acc.at[pl.ds(sid * STRIPE + j * 128, 128)])

        pltpu.sync_copy(src3_hbm.at[wid], src_v)
        pltpu.sync_copy(dst3_hbm.at[wid], dst_v)
        pltpu.sync_copy(w_hbm.at[pl.ds(base, EW)], w_v)
        plsc.subcore_barrier()

        pltpu.async_copy(h_hbm.at[src_v.at[0]], gb0, gsem0)

        @pl.loop(0, CH, step=2)
        def _(c):
            for par in (0, 1):
                cc = c + par

                @pl.when(cc < CH)
                def _():
                    gb, sb = gbufs[par], sbufs[par]
                    pltpu.make_async_copy(
                        h_hbm.at[src_v.at[cc]], gb, gsems[par]).wait()

                    @pl.when(cc + 1 < CH)
                    def _():
                        pltpu.async_copy(h_hbm.at[src_v.at[cc + 1]],
                                         gbufs[1 - par], gsems[1 - par])

                    @pl.when(cc >= 2)
                    def _():
                        pltpu.make_async_copy(
                            sb, acc.at[dst_v.at[cc]], ssems[par]).wait()

                    @pl.loop(0, 128, step=2)
                    def _(j):
                        for r in range(2):
                            jr = j + r
                            idx16 = jnp.full((16,), cc * 128 + jr, jnp.int32)
                            wv = plsc.load_gather(w_v, [idx16])
                            for t in range(4):
                                g16 = gb[jr, pl.ds(t * 16, 16)]
                                sb[jr, pl.ds(t * 16, 16)] = g16 * wv

                    pltpu.async_copy(sb, acc.at[dst_v.at[cc]], ssems[par],
                                     add=True)

        pltpu.make_async_copy(sb0, acc.at[dst_v.at[0]], ssem0).wait()
        pltpu.make_async_copy(sb1, acc.at[dst_v.at[1]], ssem1).wait()
        plsc.subcore_barrier()
        pltpu.sync_copy(acc.at[pl.ds(sid * STRIPE, STRIPE)],
                        out_hbm.at[cid].at[pl.ds(sid * STRIPE, STRIPE)])

    return kern(h64, w, src3, dst3, zeros)


def _layer_out(p_refs, den_ref, b_ref):
    den = jnp.sum(den_ref[...], axis=0)[:, None] + jnp.float32(1e-16)
    parts = [pr[0] + pr[1] for pr in p_refs]
    return jnp.maximum(jnp.concatenate(parts, axis=1) / den + b_ref[...], 0.0)


def _comb_dense_body(p_ref, den_ref, b_ref, w_ref, asrc_ref, adst_ref,
                     hA_ref, hB_ref, as_ref, ad_ref):
    x2 = _layer_out([p_ref], den_ref, b_ref)
    h = jnp.dot(x2, w_ref[...], preferred_element_type=jnp.float32)
    hA_ref[...] = h[:, :64]
    hB_ref[...] = h[:, 64:]
    as_ref[...] = jnp.sum(h * asrc_ref[...], axis=1)
    ad_ref[...] = jnp.sum(h * adst_ref[...], axis=1)


def _comb_dense(partials, den, b, W, a_src, a_dst):
    return pl.pallas_call(
        _comb_dense_body,
        grid=(GRID_N,),
        in_specs=[
            pl.BlockSpec((NC, NBLK, 64), lambda i: (0, i, 0)),
            pl.BlockSpec((NW, NBLK), lambda i: (0, i)),
            pl.BlockSpec((1, H1), lambda i: (0, 0)),
            pl.BlockSpec((H1, H2), lambda i: (0, 0)),
            pl.BlockSpec((1, H2), lambda i: (0, 0)),
            pl.BlockSpec((1, H2), lambda i: (0, 0)),
        ],
        out_specs=[pl.BlockSpec((NBLK, 64), lambda i: (i, 0))] * 2
        + [pl.BlockSpec((NBLK,), lambda i: (i,))] * 2,
        out_shape=[jax.ShapeDtypeStruct((NP, 64), jnp.float32)] * 2
        + [jax.ShapeDtypeStruct((NP,), jnp.float32)] * 2,
    )(partials, den, b.reshape(1, H1), W, a_src.reshape(1, H2),
      a_dst.reshape(1, H2))


def _comb_pool_body(pA_ref, pB_ref, den_ref, b_ref, batch_ref, wfc_ref,
                    bfc_ref, o_ref, sums, cnt):
    i = pl.program_id(0)

    @pl.when(i == 0)
    def _():
        sums[...] = jnp.zeros_like(sums)
        cnt[...] = jnp.zeros_like(cnt)

    x3 = _layer_out([pA_ref, pB_ref], den_ref, b_ref)
    bb = batch_ref[...]
    M = (lax.broadcasted_iota(jnp.int32, (NG, NBLK), 0)
         == bb[None, :]).astype(jnp.float32)
    sums[...] += jnp.dot(M, x3, preferred_element_type=jnp.float32)
    cnt[...] += jnp.sum(M, axis=1, keepdims=True)

    @pl.when(i == pl.num_programs(0) - 1)
    def _():
        pooled = sums[...] / jnp.maximum(cnt[...], 1.0)
        o_ref[...] = jnp.dot(pooled, wfc_ref[...],
                             preferred_element_type=jnp.float32) + bfc_ref[...]


def _comb_pool(pA, pB, den, b, batch_p, W_fc, b_fc):
    return pl.pallas_call(
        _comb_pool_body,
        grid=(GRID_N,),
        in_specs=[
            pl.BlockSpec((NC, NBLK, 64), lambda i: (0, i, 0)),
            pl.BlockSpec((NC, NBLK, 64), lambda i: (0, i, 0)),
            pl.BlockSpec((NW, NBLK), lambda i: (0, i)),
            pl.BlockSpec((1, H2), lambda i: (0, 0)),
            pl.BlockSpec((NBLK,), lambda i: (i,)),
            pl.BlockSpec((H2, NG), lambda i: (0, 0)),
            pl.BlockSpec((1, NG), lambda i: (0, 0)),
        ],
        out_specs=pl.BlockSpec((NG, NG), lambda i: (0, 0)),
        out_shape=jax.ShapeDtypeStruct((NG, NG), jnp.float32),
        scratch_shapes=[
            pltpu.VMEM((NG, H2), jnp.float32),
            pltpu.VMEM((NG, 1), jnp.float32),
        ],
    )(pA, pB, den, b.reshape(1, H2), batch_p, W_fc, b_fc.reshape(1, NG))


def kernel(x, edge_index, batch, W1, a1_src, a1_dst, b1, W2, a2_src, a2_dst,
           b2, W_fc, b_fc):
    loop = jnp.arange(N, dtype=edge_index.dtype)
    src = jnp.concatenate([edge_index[0], loop])
    dst = jnp.concatenate([edge_index[1], loop])
    pad = jnp.zeros((EEP - EE,), jnp.int32)
    srcp = jnp.concatenate([src, pad])
    dstp = jnp.concatenate([dst, pad])
    src3 = srcp.reshape(NW, CH, 128)
    dst3 = dstp.reshape(NW, CH, 128)
    xp = jnp.pad(x, ((0, NP - N), (0, 0)))
    batch_p = jnp.pad(batch, (0, NP - N), constant_values=NG)
    zeros = jnp.zeros((128, 64), jnp.float32)

    h1, as1, ad1 = _dense(xp, W1, a1_src, a1_dst, H1)
    w1, den1 = _edge_w(as1, ad1, srcp, dstp)
    p1 = _edge_pass(h1, w1, src3, dst3, zeros)
    hA, hB, as2, ad2 = _comb_dense(p1, den1, b1, W2, a2_src, a2_dst)
    w2, den2 = _edge_w(as2, ad2, srcp, dstp)
    pA = _edge_pass(hA, w2, src3, dst3, zeros)
    pB = _edge_pass(hB, w2, src3, dst3, zeros)
    return _comb_pool(pA, pB, den2, b2, batch_p, W_fc, b_fc)


# speedup vs baseline: 1.1954x; 1.0373x over previous
import dataclasses
import functools

import jax
import jax.numpy as jnp
from jax import lax
from jax.experimental import pallas as pl
from jax.experimental.pallas import tpu as pltpu
from jax.experimental.pallas import tpu_sc as plsc

N = 10000
NP = 10240
E = 320000
EE = E + N
NC, NS = 2, 16
NW = NC * NS
CH = 81
EW = CH * 128
EEP = EW * NW
NG = 64
F_IN = 128
H1, P1 = 64, 80
H2, P2 = 128, 144
NBLK = 2048
GRID_N = NP // NBLK
STRIPE = NP // NS

_mesh = plsc.VectorSubcoreMesh(
    core_axis_name="c", subcore_axis_name="s", num_cores=NC, num_subcores=NS)

_sc_params = pltpu.CompilerParams()
if "needs_layout_passes" in pltpu.CompilerParams.__dataclass_fields__:
    _sc_params = dataclasses.replace(_sc_params, needs_layout_passes=False)
if "use_tc_tiling_on_sc" in pltpu.CompilerParams.__dataclass_fields__:
    _sc_params = dataclasses.replace(_sc_params, use_tc_tiling_on_sc=False)


def _bf16_bits(x):
    b = jax.lax.bitcast_convert_type(x, jnp.int32)
    return (b + jnp.int32(0x7FFF) + ((b >> 16) & jnp.int32(1))) >> 16


def _interleave_bf16(h64):
    parts = []
    for g in range(2):
        blk = h64[:, 32 * g:32 * g + 32]
        lo = _bf16_bits(blk[:, :16]) & jnp.int32(0xFFFF)
        hi = _bf16_bits(blk[:, 16:])
        parts.append(lo | (hi << 16))
    return jnp.concatenate(parts, axis=1)


def _dense_body(x_ref, w_ref, asrc_ref, adst_ref, *out_refs):
    h = jnp.dot(x_ref[...], w_ref[...], preferred_element_type=jnp.float32)
    nh = len(out_refs) - 2
    for k in range(nh):
        out_refs[k][...] = h[:, k * 64:(k + 1) * 64]
    out_refs[nh][...] = jnp.sum(h * asrc_ref[...], axis=1)
    out_refs[nh + 1][...] = jnp.sum(h * adst_ref[...], axis=1)


def _dense(x, W, a_src, a_dst, H):
    F = x.shape[1]
    nh = H // 64
    return pl.pallas_call(
        _dense_body,
        grid=(GRID_N,),
        in_specs=[
            pl.BlockSpec((NBLK, F), lambda i: (i, 0)),
            pl.BlockSpec((F, H), lambda i: (0, 0)),
            pl.BlockSpec((1, H), lambda i: (0, 0)),
            pl.BlockSpec((1, H), lambda i: (0, 0)),
        ],
        out_specs=[pl.BlockSpec((NBLK, 64), lambda i: (i, 0))] * nh
        + [pl.BlockSpec((NBLK,), lambda i: (i,))] * 2,
        out_shape=[jax.ShapeDtypeStruct((NP, 64), jnp.float32)] * nh
        + [jax.ShapeDtypeStruct((NP,), jnp.float32)] * 2,
    )(x, W, a_src.reshape(1, H), a_dst.reshape(1, H))


@functools.partial(
    pl.kernel,
    out_type=(jax.ShapeDtypeStruct((EEP,), jnp.float32),
              jax.ShapeDtypeStruct((NW, NP), jnp.float32)),
    mesh=_mesh,
    scratch_types=[
        pltpu.VMEM((NP,), jnp.float32),
        pltpu.VMEM((NP,), jnp.float32),
        pltpu.VMEM((EW,), jnp.int32),
        pltpu.VMEM((EW,), jnp.int32),
        pltpu.VMEM((EW,), jnp.float32),
        pltpu.VMEM((NP,), jnp.float32),
    ],
    compiler_params=_sc_params,
)
def _edge_w(as_hbm, ad_hbm, src_hbm, dst_hbm, w_hbm, den_hbm, as_v, ad_v,
            src_v, dst_v, w_v, den_v):
    wid = lax.axis_index("s") * NC + lax.axis_index("c")
    base = wid * EW
    pltpu.sync_copy(as_hbm, as_v)
    pltpu.sync_copy(ad_hbm, ad_v)
    pltpu.sync_copy(src_hbm.at[pl.ds(base, EW)], src_v)
    pltpu.sync_copy(dst_hbm.at[pl.ds(base, EW)], dst_v)

    @pl.loop(0, NP, step=16)
    def _(k):
        den_v[pl.ds(k, 16)] = jnp.zeros((16,), jnp.float32)

    @pl.loop(0, EW, step=16)
    def _(k):
        s16 = src_v[pl.ds(k, 16)]
        d16 = dst_v[pl.ds(k, 16)]
        gs = plsc.load_gather(as_v, [s16])
        gd = plsc.load_gather(ad_v, [d16])
        e = gs + gd
        e = jnp.where(e >= 0.0, e, e * jnp.float32(0.2))
        w = jnp.exp(e)
        gi = base + k + lax.iota(jnp.int32, 16)
        w = jnp.where(gi < EE, w, jnp.float32(0.0))
        w_v[pl.ds(k, 16)] = w
        plsc.addupdate_scatter(den_v, [d16], w)

    pltpu.sync_copy(w_v, w_hbm.at[pl.ds(base, EW)])
    pltpu.sync_copy(den_v, den_hbm.at[wid])


def _edge_pass(h64, w, src3, dst3, zeros, P=64):
    @functools.partial(
        pl.kernel,
        out_type=jax.ShapeDtypeStruct((NC, NP, P), jnp.float32),
        mesh=_mesh,
        scratch_types=[
            pltpu.VMEM((CH, 128), jnp.int32),
            pltpu.VMEM((CH, 128), jnp.int32),
            pltpu.VMEM((EW,), jnp.float32),
            pltpu.VMEM((128, 64), jnp.float32),
            pltpu.VMEM((128, 64), jnp.float32),
            pltpu.VMEM((128, 64), jnp.float32),
            pltpu.VMEM((128, 64), jnp.float32),
            pltpu.VMEM_SHARED((NP, P), jnp.float32),
            pltpu.SemaphoreType.DMA,
            pltpu.SemaphoreType.DMA,
            pltpu.SemaphoreType.DMA,
            pltpu.SemaphoreType.DMA,
            pltpu.SemaphoreType.DMA,
            pltpu.SemaphoreType.DMA,
            pltpu.SemaphoreType.DMA,
            pltpu.SemaphoreType.DMA,
        ],
        compiler_params=_sc_params,
    )
    def kern(h_hbm, w_hbm, src3_hbm, dst3_hbm, z_hbm, out_hbm,
             src_v, dst_v, w_v, b0, b1, b2, b3, acc,
             gsem0, gsem1, gsem2, gsem3, ssem0, ssem1, ssem2, ssem3):
        cid = lax.axis_index("c")
        sid = lax.axis_index("s")
        wid = sid * NC + cid
        base = wid * EW
        bufs = (b0, b1, b2, b3)
        gsems = (gsem0, gsem1, gsem2, gsem3)
        ssems = (ssem0, ssem1, ssem2, ssem3)

        @pl.loop(0, STRIPE // 128)
        def _(j):
            pltpu.sync_copy(z_hbm, acc.at[pl.ds(sid * STRIPE + j * 128, 128)])

        pltpu.sync_copy(src3_hbm.at[wid], src_v)
        pltpu.sync_copy(dst3_hbm.at[wid], dst_v)
        pltpu.sync_copy(w_hbm.at[pl.ds(base, EW)], w_v)
        plsc.subcore_barrier()

        pltpu.async_copy(h_hbm.at[src_v.at[0]], b0, gsem0)
        pltpu.async_copy(h_hbm.at[src_v.at[1]], b1, gsem1)

        @pl.loop(0, 84, step=4)
        def _(c):
            for par in (0, 1, 2, 3):
                cc = c + par

                @pl.when(cc < CH)
                def _():
                    gb = bufs[par]
                    nb = bufs[(par + 2) % 4]
                    pltpu.make_async_copy(
                        h_hbm.at[src_v.at[cc]], gb, gsems[par]).wait()

                    @pl.when(cc + 2 < CH)
                    def _():
                        @pl.when(cc >= 2)
                        def _():
                            pltpu.make_async_copy(
                                nb, acc.at[dst_v.at[cc]],
                                ssems[(par + 2) % 4]).wait()

                        pltpu.async_copy(h_hbm.at[src_v.at[cc + 2]], nb,
                                         gsems[(par + 2) % 4])

                    @pl.loop(0, 128, step=2)
                    def _(j):
                        for r in range(2):
                            jr = j + r
                            idx16 = jnp.full((16,), cc * 128 + jr, jnp.int32)
                            wv = plsc.load_gather(w_v, [idx16])
                            for t in range(4):
                                gb[jr, pl.ds(t * 16, 16)] = (
                                    gb[jr, pl.ds(t * 16, 16)] * wv)

                    pltpu.async_copy(gb, acc.at[dst_v.at[cc]], ssems[par],
                                     add=True)

        for par in (0, 1, 2, 3):
            pltpu.make_async_copy(bufs[par], acc.at[dst_v.at[0]],
                                  ssems[par]).wait()
        plsc.subcore_barrier()
        pltpu.sync_copy(acc.at[pl.ds(sid * STRIPE, STRIPE)],
                        out_hbm.at[cid].at[pl.ds(sid * STRIPE, STRIPE)])

    return kern(h64, w, src3, dst3, zeros)


def _layer_out(p_refs, den_ref, b_ref):
    den = jnp.sum(den_ref[...], axis=0)[:, None] + jnp.float32(1e-16)
    parts = [pr[0] + pr[1] for pr in p_refs]
    return jnp.maximum(jnp.concatenate(parts, axis=1) / den + b_ref[...], 0.0)


def _comb_dense_body(p_ref, den_ref, b_ref, w_ref, asrc_ref, adst_ref,
                     hA_ref, hB_ref, as_ref, ad_ref):
    x2 = _layer_out([p_ref], den_ref, b_ref)
    h = jnp.dot(x2, w_ref[...], preferred_element_type=jnp.float32)
    hA_ref[...] = h[:, :64]
    hB_ref[...] = h[:, 64:]
    as_ref[...] = jnp.sum(h * asrc_ref[...], axis=1)
    ad_ref[...] = jnp.sum(h * adst_ref[...], axis=1)


def _comb_dense(partials, den, b, W, a_src, a_dst):
    return pl.pallas_call(
        _comb_dense_body,
        grid=(GRID_N,),
        in_specs=[
            pl.BlockSpec((NC, NBLK, 64), lambda i: (0, i, 0)),
            pl.BlockSpec((NW, NBLK), lambda i: (0, i)),
            pl.BlockSpec((1, H1), lambda i: (0, 0)),
            pl.BlockSpec((H1, H2), lambda i: (0, 0)),
            pl.BlockSpec((1, H2), lambda i: (0, 0)),
            pl.BlockSpec((1, H2), lambda i: (0, 0)),
        ],
        out_specs=[pl.BlockSpec((NBLK, 64), lambda i: (i, 0))] * 2
        + [pl.BlockSpec((NBLK,), lambda i: (i,))] * 2,
        out_shape=[jax.ShapeDtypeStruct((NP, 64), jnp.float32)] * 2
        + [jax.ShapeDtypeStruct((NP,), jnp.float32)] * 2,
    )(partials, den, b.reshape(1, H1), W, a_src.reshape(1, H2),
      a_dst.reshape(1, H2))


def _comb_pool_body(pA_ref, pB_ref, den_ref, b_ref, batch_ref, wfc_ref,
                    bfc_ref, o_ref, sums, cnt):
    i = pl.program_id(0)

    @pl.when(i == 0)
    def _():
        sums[...] = jnp.zeros_like(sums)
        cnt[...] = jnp.zeros_like(cnt)

    x3 = _layer_out([pA_ref, pB_ref], den_ref, b_ref)
    bb = batch_ref[...]
    M = (lax.broadcasted_iota(jnp.int32, (NG, NBLK), 0)
         == bb[None, :]).astype(jnp.float32)
    sums[...] += jnp.dot(M, x3, preferred_element_type=jnp.float32)
    cnt[...] += jnp.sum(M, axis=1, keepdims=True)

    @pl.when(i == pl.num_programs(0) - 1)
    def _():
        pooled = sums[...] / jnp.maximum(cnt[...], 1.0)
        o_ref[...] = jnp.dot(pooled, wfc_ref[...],
                             preferred_element_type=jnp.float32) + bfc_ref[...]


def _comb_pool(pA, pB, den, b, batch_p, W_fc, b_fc):
    return pl.pallas_call(
        _comb_pool_body,
        grid=(GRID_N,),
        in_specs=[
            pl.BlockSpec((NC, NBLK, 64), lambda i: (0, i, 0)),
            pl.BlockSpec((NC, NBLK, 64), lambda i: (0, i, 0)),
            pl.BlockSpec((NW, NBLK), lambda i: (0, i)),
            pl.BlockSpec((1, H2), lambda i: (0, 0)),
            pl.BlockSpec((NBLK,), lambda i: (i,)),
            pl.BlockSpec((H2, NG), lambda i: (0, 0)),
            pl.BlockSpec((1, NG), lambda i: (0, 0)),
        ],
        out_specs=pl.BlockSpec((NG, NG), lambda i: (0, 0)),
        out_shape=jax.ShapeDtypeStruct((NG, NG), jnp.float32),
        scratch_shapes=[
            pltpu.VMEM((NG, H2), jnp.float32),
            pltpu.VMEM((NG, 1), jnp.float32),
        ],
    )(pA, pB, den, b.reshape(1, H2), batch_p, W_fc, b_fc.reshape(1, NG))


def kernel(x, edge_index, batch, W1, a1_src, a1_dst, b1, W2, a2_src, a2_dst,
           b2, W_fc, b_fc):
    loop = jnp.arange(N, dtype=edge_index.dtype)
    src = jnp.concatenate([edge_index[0], loop])
    dst = jnp.concatenate([edge_index[1], loop])
    pad = jnp.zeros((EEP - EE,), jnp.int32)
    srcp = jnp.concatenate([src, pad])
    dstp = jnp.concatenate([dst, pad])
    src3 = srcp.reshape(NW, CH, 128)
    dst3 = dstp.reshape(NW, CH, 128)
    xp = jnp.pad(x, ((0, NP - N), (0, 0)))
    batch_p = jnp.pad(batch, (0, NP - N), constant_values=NG)
    zeros = jnp.zeros((128, 64), jnp.float32)

    h1, as1, ad1 = _dense(xp, W1, a1_src, a1_dst, H1)
    w1, den1 = _edge_w(as1, ad1, srcp, dstp)
    p1 = _edge_pass(h1, w1, src3, dst3, zeros)
    hA, hB, as2, ad2 = _comb_dense(p1, den1, b1, W2, a2_src, a2_dst)
    w2, den2 = _edge_w(as2, ad2, srcp, dstp)
    pA = _edge_pass(hA, w2, src3, dst3, zeros)
    pB = _edge_pass(hB, w2, src3, dst3, zeros)
    return _comb_pool(pA, pB, den2, b2, batch_p, W_fc, b_fc)


# speedup vs baseline: 1.2144x; 1.0158x over previous
import dataclasses
import functools

import jax
import jax.numpy as jnp
from jax import lax
from jax.experimental import pallas as pl
from jax.experimental.pallas import tpu as pltpu
from jax.experimental.pallas import tpu_sc as plsc

N = 10000
NP = 10240
E = 320000
EE = E + N
NC, NS = 2, 16
NW = NC * NS
CH = 81
EW = CH * 128
EEP = EW * NW
NG = 64
F_IN = 128
H1, P1 = 64, 80
H2, P2 = 128, 144
NBLK = 2048
GRID_N = NP // NBLK
STRIPE = NP // NS

_mesh = plsc.VectorSubcoreMesh(
    core_axis_name="c", subcore_axis_name="s", num_cores=NC, num_subcores=NS)

_sc_params = pltpu.CompilerParams()
if "needs_layout_passes" in pltpu.CompilerParams.__dataclass_fields__:
    _sc_params = dataclasses.replace(_sc_params, needs_layout_passes=False)
if "use_tc_tiling_on_sc" in pltpu.CompilerParams.__dataclass_fields__:
    _sc_params = dataclasses.replace(_sc_params, use_tc_tiling_on_sc=False)


def _bf16_bits(x):
    b = jax.lax.bitcast_convert_type(x, jnp.int32)
    return (b + jnp.int32(0x7FFF) + ((b >> 16) & jnp.int32(1))) >> 16


def _interleave_bf16(h64):
    parts = []
    for g in range(2):
        blk = h64[:, 32 * g:32 * g + 32]
        lo = _bf16_bits(blk[:, :16]) & jnp.int32(0xFFFF)
        hi = _bf16_bits(blk[:, 16:])
        parts.append(lo | (hi << 16))
    return jnp.concatenate(parts, axis=1)


def _dense_body(x_ref, w_ref, asrc_ref, adst_ref, *out_refs):
    h = jnp.dot(x_ref[...], w_ref[...], preferred_element_type=jnp.float32)
    nh = len(out_refs) - 2
    for k in range(nh):
        out_refs[k][...] = h[:, k * 64:(k + 1) * 64]
    out_refs[nh][...] = jnp.sum(h * asrc_ref[...], axis=1)
    out_refs[nh + 1][...] = jnp.sum(h * adst_ref[...], axis=1)


def _dense(x, W, a_src, a_dst, H):
    F = x.shape[1]
    nh = H // 64
    return pl.pallas_call(
        _dense_body,
        grid=(GRID_N,),
        in_specs=[
            pl.BlockSpec((NBLK, F), lambda i: (i, 0)),
            pl.BlockSpec((F, H), lambda i: (0, 0)),
            pl.BlockSpec((1, H), lambda i: (0, 0)),
            pl.BlockSpec((1, H), lambda i: (0, 0)),
        ],
        out_specs=[pl.BlockSpec((NBLK, 64), lambda i: (i, 0))] * nh
        + [pl.BlockSpec((NBLK,), lambda i: (i,))] * 2,
        out_shape=[jax.ShapeDtypeStruct((NP, 64), jnp.float32)] * nh
        + [jax.ShapeDtypeStruct((NP,), jnp.float32)] * 2,
    )(x, W, a_src.reshape(1, H), a_dst.reshape(1, H))


@functools.partial(
    pl.kernel,
    out_type=(jax.ShapeDtypeStruct((EEP,), jnp.float32),
              jax.ShapeDtypeStruct((NW, NP), jnp.float32)),
    mesh=_mesh,
    scratch_types=[
        pltpu.VMEM((NP,), jnp.float32),
        pltpu.VMEM((NP,), jnp.float32),
        pltpu.VMEM((EW,), jnp.int32),
        pltpu.VMEM((EW,), jnp.int32),
        pltpu.VMEM((EW,), jnp.float32),
        pltpu.VMEM((NP,), jnp.float32),
    ],
    compiler_params=_sc_params,
)
def _edge_w(as_hbm, ad_hbm, src_hbm, dst_hbm, w_hbm, den_hbm, as_v, ad_v,
            src_v, dst_v, w_v, den_v):
    wid = lax.axis_index("s") * NC + lax.axis_index("c")
    base = wid * EW
    pltpu.sync_copy(as_hbm, as_v)
    pltpu.sync_copy(ad_hbm, ad_v)
    pltpu.sync_copy(src_hbm.at[pl.ds(base, EW)], src_v)
    pltpu.sync_copy(dst_hbm.at[pl.ds(base, EW)], dst_v)

    @pl.loop(0, NP, step=16)
    def _(k):
        den_v[pl.ds(k, 16)] = jnp.zeros((16,), jnp.float32)

    @pl.loop(0, EW, step=16)
    def _(k):
        s16 = src_v[pl.ds(k, 16)]
        d16 = dst_v[pl.ds(k, 16)]
        gs = plsc.load_gather(as_v, [s16])
        gd = plsc.load_gather(ad_v, [d16])
        e = gs + gd
        e = jnp.where(e >= 0.0, e, e * jnp.float32(0.2))
        w = jnp.exp(e)
        gi = base + k + lax.iota(jnp.int32, 16)
        w = jnp.where(gi < EE, w, jnp.float32(0.0))
        w_v[pl.ds(k, 16)] = w
        plsc.addupdate_scatter(den_v, [d16], w)

    pltpu.sync_copy(w_v, w_hbm.at[pl.ds(base, EW)])
    pltpu.sync_copy(den_v, den_hbm.at[wid])


def _edge_pass(h64, w, src3, dst3, zeros, P=64):
    @functools.partial(
        pl.kernel,
        out_type=jax.ShapeDtypeStruct((NC, NP, P), jnp.float32),
        mesh=_mesh,
        scratch_types=[
            pltpu.VMEM((CH, 128), jnp.int32),
            pltpu.VMEM((CH, 128), jnp.int32),
            pltpu.VMEM((EW,), jnp.float32),
        ]
        + [pltpu.VMEM((128, 64), jnp.float32)] * 6
        + [pltpu.VMEM_SHARED((NP, P), jnp.float32)]
        + [pltpu.SemaphoreType.DMA] * 13,
        compiler_params=_sc_params,
    )
    def kern(h_hbm, w_hbm, src3_hbm, dst3_hbm, z_hbm, out_hbm,
             src_v, dst_v, w_v, b0, b1, b2, b3, b4, b5, acc, *sems):
        cid = lax.axis_index("c")
        sid = lax.axis_index("s")
        wid = sid * NC + cid
        base = wid * EW
        bufs = (b0, b1, b2, b3, b4, b5)
        gsems = sems[0:6]
        ssems = sems[6:12]
        zsem = sems[12]

        for j in range(STRIPE // 128):
            pltpu.async_copy(
                z_hbm, acc.at[pl.ds(sid * STRIPE + j * 128, 128)], zsem)

        pltpu.sync_copy(src3_hbm.at[wid], src_v)
        pltpu.sync_copy(dst3_hbm.at[wid], dst_v)
        pltpu.sync_copy(w_hbm.at[pl.ds(base, EW)], w_v)
        for j in range(STRIPE // 128):
            pltpu.make_async_copy(
                z_hbm, acc.at[pl.ds(sid * STRIPE + j * 128, 128)],
                zsem).wait()
        plsc.subcore_barrier()

        for p in range(4):
            pltpu.async_copy(h_hbm.at[src_v.at[p]], bufs[p], gsems[p])

        @pl.loop(0, 84, step=6)
        def _(c):
            for par in (0, 1, 2, 3, 4, 5):
                cc = c + par

                @pl.when(cc < CH)
                def _():
                    gb = bufs[par]
                    nb = bufs[(par + 4) % 6]
                    pltpu.make_async_copy(
                        h_hbm.at[src_v.at[cc]], gb, gsems[par]).wait()

                    @pl.when(cc + 4 < CH)
                    def _():
                        @pl.when(cc >= 2)
                        def _():
                            pltpu.make_async_copy(
                                nb, acc.at[dst_v.at[cc]],
                                ssems[(par + 4) % 6]).wait()

                        pltpu.async_copy(h_hbm.at[src_v.at[cc + 4]], nb,
                                         gsems[(par + 4) % 6])

                    @pl.loop(0, 128, step=2)
                    def _(j):
                        for r in range(2):
                            jr = j + r
                            idx16 = jnp.full((16,), cc * 128 + jr, jnp.int32)
                            wv = plsc.load_gather(w_v, [idx16])
                            for t in range(4):
                                gb[jr, pl.ds(t * 16, 16)] = (
                                    gb[jr, pl.ds(t * 16, 16)] * wv)

                    pltpu.async_copy(gb, acc.at[dst_v.at[cc]], ssems[par],
                                     add=True)

        for par in range(6):
            pltpu.make_async_copy(bufs[par], acc.at[dst_v.at[0]],
                                  ssems[par]).wait()
        plsc.subcore_barrier()
        pltpu.sync_copy(acc.at[pl.ds(sid * STRIPE, STRIPE)],
                        out_hbm.at[cid].at[pl.ds(sid * STRIPE, STRIPE)])

    return kern(h64, w, src3, dst3, zeros)


def _layer_out(p_refs, den_ref, b_ref):
    den = jnp.sum(den_ref[...], axis=0)[:, None] + jnp.float32(1e-16)
    parts = [pr[0] + pr[1] for pr in p_refs]
    return jnp.maximum(jnp.concatenate(parts, axis=1) / den + b_ref[...], 0.0)


def _comb_dense_body(p_ref, den_ref, b_ref, w_ref, asrc_ref, adst_ref,
                     hA_ref, hB_ref, as_ref, ad_ref):
    x2 = _layer_out([p_ref], den_ref, b_ref)
    h = jnp.dot(x2, w_ref[...], preferred_element_type=jnp.float32)
    hA_ref[...] = h[:, :64]
    hB_ref[...] = h[:, 64:]
    as_ref[...] = jnp.sum(h * asrc_ref[...], axis=1)
    ad_ref[...] = jnp.sum(h * adst_ref[...], axis=1)


def _comb_dense(partials, den, b, W, a_src, a_dst):
    return pl.pallas_call(
        _comb_dense_body,
        grid=(GRID_N,),
        in_specs=[
            pl.BlockSpec((NC, NBLK, 64), lambda i: (0, i, 0)),
            pl.BlockSpec((NW, NBLK), lambda i: (0, i)),
            pl.BlockSpec((1, H1), lambda i: (0, 0)),
            pl.BlockSpec((H1, H2), lambda i: (0, 0)),
            pl.BlockSpec((1, H2), lambda i: (0, 0)),
            pl.BlockSpec((1, H2), lambda i: (0, 0)),
        ],
        out_specs=[pl.BlockSpec((NBLK, 64), lambda i: (i, 0))] * 2
        + [pl.BlockSpec((NBLK,), lambda i: (i,))] * 2,
        out_shape=[jax.ShapeDtypeStruct((NP, 64), jnp.float32)] * 2
        + [jax.ShapeDtypeStruct((NP,), jnp.float32)] * 2,
    )(partials, den, b.reshape(1, H1), W, a_src.reshape(1, H2),
      a_dst.reshape(1, H2))


def _comb_pool_body(pA_ref, pB_ref, den_ref, b_ref, batch_ref, wfc_ref,
                    bfc_ref, o_ref, sums, cnt):
    i = pl.program_id(0)

    @pl.when(i == 0)
    def _():
        sums[...] = jnp.zeros_like(sums)
        cnt[...] = jnp.zeros_like(cnt)

    x3 = _layer_out([pA_ref, pB_ref], den_ref, b_ref)
    bb = batch_ref[...]
    M = (lax.broadcasted_iota(jnp.int32, (NG, NBLK), 0)
         == bb[None, :]).astype(jnp.float32)
    sums[...] += jnp.dot(M, x3, preferred_element_type=jnp.float32)
    cnt[...] += jnp.sum(M, axis=1, keepdims=True)

    @pl.when(i == pl.num_programs(0) - 1)
    def _():
        pooled = sums[...] / jnp.maximum(cnt[...], 1.0)
        o_ref[...] = jnp.dot(pooled, wfc_ref[...],
                             preferred_element_type=jnp.float32) + bfc_ref[...]


def _comb_pool(pA, pB, den, b, batch_p, W_fc, b_fc):
    return pl.pallas_call(
        _comb_pool_body,
        grid=(GRID_N,),
        in_specs=[
            pl.BlockSpec((NC, NBLK, 64), lambda i: (0, i, 0)),
            pl.BlockSpec((NC, NBLK, 64), lambda i: (0, i, 0)),
            pl.BlockSpec((NW, NBLK), lambda i: (0, i)),
            pl.BlockSpec((1, H2), lambda i: (0, 0)),
            pl.BlockSpec((NBLK,), lambda i: (i,)),
            pl.BlockSpec((H2, NG), lambda i: (0, 0)),
            pl.BlockSpec((1, NG), lambda i: (0, 0)),
        ],
        out_specs=pl.BlockSpec((NG, NG), lambda i: (0, 0)),
        out_shape=jax.ShapeDtypeStruct((NG, NG), jnp.float32),
        scratch_shapes=[
            pltpu.VMEM((NG, H2), jnp.float32),
            pltpu.VMEM((NG, 1), jnp.float32),
        ],
    )(pA, pB, den, b.reshape(1, H2), batch_p, W_fc, b_fc.reshape(1, NG))


def kernel(x, edge_index, batch, W1, a1_src, a1_dst, b1, W2, a2_src, a2_dst,
           b2, W_fc, b_fc):
    loop = jnp.arange(N, dtype=edge_index.dtype)
    src = jnp.concatenate([edge_index[0], loop])
    dst = jnp.concatenate([edge_index[1], loop])
    pad = jnp.zeros((EEP - EE,), jnp.int32)
    srcp = jnp.concatenate([src, pad])
    dstp = jnp.concatenate([dst, pad])
    src3 = srcp.reshape(NW, CH, 128)
    dst3 = dstp.reshape(NW, CH, 128)
    xp = jnp.pad(x, ((0, NP - N), (0, 0)))
    batch_p = jnp.pad(batch, (0, NP - N), constant_values=NG)
    zeros = jnp.zeros((128, 64), jnp.float32)

    h1, as1, ad1 = _dense(xp, W1, a1_src, a1_dst, H1)
    w1, den1 = _edge_w(as1, ad1, srcp, dstp)
    p1 = _edge_pass(h1, w1, src3, dst3, zeros)
    hA, hB, as2, ad2 = _comb_dense(p1, den1, b1, W2, a2_src, a2_dst)
    w2, den2 = _edge_w(as2, ad2, srcp, dstp)
    pA = _edge_pass(hA, w2, src3, dst3, zeros)
    pB = _edge_pass(hB, w2, src3, dst3, zeros)
    return _comb_pool(pA, pB, den2, b2, batch_p, W_fc, b_fc)
